# baseline (speedup 1.0000x reference)
import functools

import jax
import jax.numpy as jnp
from jax import lax
from jax.experimental import pallas as pl
from jax.experimental.pallas import tpu as pltpu
from jax.experimental.pallas import tpu_sc as plsc


def _table_body(ee_ref, ec_ref, cwt_ref, out_ref):
    out_ref[...] = ee_ref[...] + jnp.dot(
        ec_ref[...], cwt_ref[...], preferred_element_type=jnp.float32
    )


def _compute_table(element_embedding, electron_config, config_weight):
    Zmax, F = element_embedding.shape
    return pl.pallas_call(
        _table_body,
        out_shape=jax.ShapeDtypeStruct((Zmax, F), jnp.float32),
    )(element_embedding, electron_config, config_weight.T)


_HC = 2
_NH = 2


@functools.lru_cache(maxsize=None)
def _make_gather(n_rows, n_chunks_w, ch, F, Zmax, NC, NS):
    mesh = plsc.VectorSubcoreMesh(core_axis_name="c", subcore_axis_name="s")
    hc, nh = _HC, _NH
    n_groups = n_chunks_w // hc
    n_outer = n_groups // nh

    @functools.partial(
        pl.kernel,
        mesh=mesh,
        out_type=jax.ShapeDtypeStruct((n_rows, F), jnp.float32),
        scratch_types=[
            pltpu.VMEM((n_chunks_w, ch), jnp.int32),
            pltpu.VMEM_SHARED((Zmax, F), jnp.float32),
            pltpu.VMEM((nh, hc * ch, F), jnp.float32),
        ]
        + [pltpu.SemaphoreType.DMA] * (2 * nh),
    )
    def gather(table_hbm, idx_hbm, out_hbm, idx_v, table_v, rows_v, *sems):
        gsem, wsem = sems[:nh], sems[nh:]
        wid = lax.axis_index("s") * NC + lax.axis_index("c")
        row0 = wid * n_chunks_w
        @pl.when(lax.axis_index("s") == 0)
        def _():
            pltpu.sync_copy(table_hbm, table_v)

        pltpu.sync_copy(idx_hbm.at[pl.ds(row0, n_chunks_w)], idx_v)
        plsc.subcore_barrier()

        def fire_gathers(p, h):
            for k in range(hc):
                pltpu.async_copy(
                    table_v.at[idx_v.at[p * hc + k]],
                    rows_v.at[h, pl.ds(k * ch, ch)],
                    gsem[h],
                )

        def wait_gathers(p, h):
            for k in range(hc):
                pltpu.make_async_copy(
                    table_v.at[idx_v.at[p * hc + k]],
                    rows_v.at[h, pl.ds(k * ch, ch)],
                    gsem[h],
                ).wait()

        def write_cp(p, h):
            return pltpu.make_async_copy(
                rows_v.at[h],
                out_hbm.at[pl.ds((row0 + p * hc) * ch, hc * ch)],
                wsem[h],
            )

        for h in range(nh):
            fire_gathers(h, h)

        def body(g, carry):
            for h in range(nh):
                p = g * nh + h
                wait_gathers(p, h)
                pltpu.async_copy(
                    rows_v.at[h],
                    out_hbm.at[pl.ds((row0 + p * hc) * ch, hc * ch)],
                    wsem[h],
                )
            for h in range(nh):
                p = g * nh + h
                write_cp(p, h).wait()

                @pl.when(g + 1 < n_outer)
                def _():
                    fire_gathers(p + nh, h)

            return carry

        lax.fori_loop(0, n_outer, body, 0)

    return gather


def kernel(Z, element_embedding, config_weight, electron_config):
    B, N = Z.shape
    Zmax, F = element_embedding.shape
    table = _compute_table(element_embedding, electron_config, config_weight)

    info = plsc.get_sparse_core_info()
    NC, NS = info.num_cores, info.num_subcores
    NW = NC * NS

    ch = N
    n_chunks = B
    n_chunks_w = n_chunks // NW

    idx = Z.astype(jnp.int32)
    out = _make_gather(B * N, n_chunks_w, ch, F, Zmax, NC, NS)(table, idx)
    return out.reshape(B, N, F)
